# baseline (device time: 77807 ns/iter reference)
import jax
import jax.numpy as jnp
from jax import lax
from jax.experimental import pallas as pl
from jax.experimental.pallas import tpu as pltpu

N_DEV = 4


def kernel(x, w_mat):
    x = x.astype(jnp.bfloat16)
    w_mat = w_mat.astype(jnp.bfloat16)

    m_per, k = x.shape
    n = w_mat.shape[1]
    n_per = n // N_DEV
    m_glob = N_DEV * m_per

    def body(x_ref, w_ref, out_ref, send_buf, recv_buf, send_sems, recv_sems):
        my = lax.axis_index("i")

        barrier_sem = pltpu.get_barrier_semaphore()
        for d in range(1, N_DEV):
            pl.semaphore_signal(
                barrier_sem,
                inc=1,
                device_id=((my + d) % N_DEV,),
                device_id_type=pl.DeviceIdType.MESH,
            )
        pl.semaphore_wait(barrier_sem, N_DEV - 1)

        for c in range(N_DEV):
            y_c = jnp.dot(
                x_ref[...],
                w_ref[:, c * n_per:(c + 1) * n_per],
                preferred_element_type=jnp.float32,
            )

            @pl.when(my == c)
            def _(y_c=y_c, c=c):
                out_ref[pl.ds(c * m_per, m_per), :] = y_c

            @pl.when(my != c)
            def _(y_c=y_c, c=c):
                send_buf[c, :, :] = y_c.astype(jnp.bfloat16)
                rdma = pltpu.make_async_remote_copy(
                    src_ref=send_buf.at[c],
                    dst_ref=recv_buf.at[my],
                    send_sem=send_sems.at[c],
                    recv_sem=recv_sems.at[my],
                    device_id=(c,),
                    device_id_type=pl.DeviceIdType.MESH,
                )
                rdma.start()

        for s in range(N_DEV):
            @pl.when(my != s)
            def _(s=s):
                recv = pltpu.make_async_remote_copy(
                    src_ref=send_buf.at[s],
                    dst_ref=recv_buf.at[s],
                    send_sem=send_sems.at[s],
                    recv_sem=recv_sems.at[s],
                    device_id=(s,),
                    device_id_type=pl.DeviceIdType.MESH,
                )
                recv.wait_recv()
                out_ref[pl.ds(s * m_per, m_per), :] = recv_buf[s, :, :].astype(
                    jnp.float32
                )

        for c in range(N_DEV):
            @pl.when(my != c)
            def _(c=c):
                send = pltpu.make_async_remote_copy(
                    src_ref=send_buf.at[c],
                    dst_ref=recv_buf.at[my],
                    send_sem=send_sems.at[c],
                    recv_sem=recv_sems.at[my],
                    device_id=(c,),
                    device_id_type=pl.DeviceIdType.MESH,
                )
                send.wait_send()

    return pl.pallas_call(
        body,
        out_shape=jax.ShapeDtypeStruct((m_glob, n_per), jnp.float32),
        in_specs=[
            pl.BlockSpec(memory_space=pltpu.VMEM),
            pl.BlockSpec(memory_space=pltpu.VMEM),
        ],
        out_specs=pl.BlockSpec(memory_space=pltpu.VMEM),
        scratch_shapes=[
            pltpu.VMEM((N_DEV, m_per, n_per), jnp.bfloat16),
            pltpu.VMEM((N_DEV, m_per, n_per), jnp.bfloat16),
            pltpu.SemaphoreType.DMA((N_DEV,)),
            pltpu.SemaphoreType.DMA((N_DEV,)),
        ],
        compiler_params=pltpu.CompilerParams(collective_id=0),
    )(x, w_mat)


# device time: 75787 ns/iter; 1.0267x vs baseline; 1.0267x over previous
import jax
import jax.numpy as jnp
from jax import lax
from jax.experimental import pallas as pl
from jax.experimental.pallas import tpu as pltpu

N_DEV = 4
X_CHUNKS = 4
H_PER_TILE = 2


def kernel(x, w_mat):
    m_per, k = x.shape
    n = w_mat.shape[1]
    n_per = n // N_DEV
    n_half = n_per // H_PER_TILE
    m_glob = N_DEV * m_per
    mx = m_per // X_CHUNKS
    n_halves = N_DEV * H_PER_TILE

    def body(x_hbm, w_hbm, out_ref,
             xf_buf, x_bf, wf_buf, w_bf, send_buf, recv_buf,
             x_sems, w_sems, send_sems, recv_sems):
        my = lax.axis_index("i")

        x_dmas = [
            pltpu.make_async_copy(
                x_hbm.at[pl.ds(ci * mx, mx), :], xf_buf.at[ci % 2],
                x_sems.at[ci % 2],
            )
            for ci in range(X_CHUNKS)
        ]
        w_dmas = [
            pltpu.make_async_copy(
                w_hbm.at[:, pl.ds(h * n_half, n_half)], wf_buf.at[h % 2],
                w_sems.at[h % 2],
            )
            for h in range(n_halves)
        ]
        x_dmas[0].start()
        x_dmas[1].start()
        w_dmas[0].start()
        w_dmas[1].start()

        barrier_sem = pltpu.get_barrier_semaphore()
        for d in range(1, N_DEV):
            pl.semaphore_signal(
                barrier_sem,
                inc=1,
                device_id=((my + d) % N_DEV,),
                device_id_type=pl.DeviceIdType.MESH,
            )
        pl.semaphore_wait(barrier_sem, N_DEV - 1)

        for ci in range(X_CHUNKS):
            x_dmas[ci].wait()
            x_bf[pl.ds(ci * mx, mx), :] = xf_buf[ci % 2].astype(jnp.bfloat16)
            if ci + 2 < X_CHUNKS:
                x_dmas[ci + 2].start()

        for h in range(n_halves):
            c, hh = divmod(h, H_PER_TILE)
            w_dmas[h].wait()
            w_bf[h % 2] = wf_buf[h % 2].astype(jnp.bfloat16)
            if h + 2 < n_halves:
                w_dmas[h + 2].start()

            y_h = jnp.dot(
                x_bf[...], w_bf[h % 2],
                preferred_element_type=jnp.float32,
            )

            @pl.when(my == c)
            def _(y_h=y_h, c=c, hh=hh):
                out_ref[pl.ds(c * m_per, m_per),
                        hh * n_half:(hh + 1) * n_half] = y_h

            @pl.when(my != c)
            def _(y_h=y_h, c=c, hh=hh):
                send_buf[c, hh] = y_h.astype(jnp.bfloat16)
                rdma = pltpu.make_async_remote_copy(
                    src_ref=send_buf.at[c, hh],
                    dst_ref=recv_buf.at[my, hh],
                    send_sem=send_sems.at[H_PER_TILE * c + hh],
                    recv_sem=recv_sems.at[H_PER_TILE * my + hh],
                    device_id=(c,),
                    device_id_type=pl.DeviceIdType.MESH,
                )
                rdma.start()

        for s in range(N_DEV):
            for hh in range(H_PER_TILE):
                @pl.when(my != s)
                def _(s=s, hh=hh):
                    recv = pltpu.make_async_remote_copy(
                        src_ref=send_buf.at[s, hh],
                        dst_ref=recv_buf.at[s, hh],
                        send_sem=send_sems.at[H_PER_TILE * s + hh],
                        recv_sem=recv_sems.at[H_PER_TILE * s + hh],
                        device_id=(s,),
                        device_id_type=pl.DeviceIdType.MESH,
                    )
                    recv.wait_recv()
                    out_ref[pl.ds(s * m_per, m_per),
                            hh * n_half:(hh + 1) * n_half] = (
                        recv_buf[s, hh].astype(jnp.float32))

        for c in range(N_DEV):
            for hh in range(H_PER_TILE):
                @pl.when(my != c)
                def _(c=c, hh=hh):
                    send = pltpu.make_async_remote_copy(
                        src_ref=send_buf.at[c, hh],
                        dst_ref=recv_buf.at[my, hh],
                        send_sem=send_sems.at[H_PER_TILE * c + hh],
                        recv_sem=recv_sems.at[H_PER_TILE * my + hh],
                        device_id=(c,),
                        device_id_type=pl.DeviceIdType.MESH,
                    )
                    send.wait_send()

    return pl.pallas_call(
        body,
        out_shape=jax.ShapeDtypeStruct((m_glob, n_per), jnp.float32),
        in_specs=[
            pl.BlockSpec(memory_space=pltpu.MemorySpace.HBM),
            pl.BlockSpec(memory_space=pltpu.MemorySpace.HBM),
        ],
        out_specs=pl.BlockSpec(memory_space=pltpu.VMEM),
        scratch_shapes=[
            pltpu.VMEM((2, mx, k), jnp.float32),
            pltpu.VMEM((m_per, k), jnp.bfloat16),
            pltpu.VMEM((2, k, n_half), jnp.float32),
            pltpu.VMEM((2, k, n_half), jnp.bfloat16),
            pltpu.VMEM((N_DEV, H_PER_TILE, m_per, n_half), jnp.bfloat16),
            pltpu.VMEM((N_DEV, H_PER_TILE, m_per, n_half), jnp.bfloat16),
            pltpu.SemaphoreType.DMA((2,)),
            pltpu.SemaphoreType.DMA((2,)),
            pltpu.SemaphoreType.DMA((N_DEV * H_PER_TILE,)),
            pltpu.SemaphoreType.DMA((N_DEV * H_PER_TILE,)),
        ],
        compiler_params=pltpu.CompilerParams(
            collective_id=0,
            vmem_limit_bytes=100 * 1024 * 1024,
        ),
    )(x, w_mat)


# device time: 52302 ns/iter; 1.4876x vs baseline; 1.4490x over previous
import os

import jax
import jax.numpy as jnp
from jax import lax
from jax.experimental import pallas as pl
from jax.experimental.pallas import tpu as pltpu

_KVAR = os.environ.get("KVAR", "full")
_DO_COMPUTE = _KVAR in ("full", "nocomm")
_DO_COMM = _KVAR != "nocomm"
_COMM_STEPS = {
    "nodiag": (0, 2),
    "diagonly": (1,),
}.get(_KVAR, (0, 1, 2))

N_DEV = 4
X_CHUNKS = int(os.environ.get("KXCHUNKS", "4"))
_UNIFORM_ORDER = os.environ.get("KUNIFORM", "0") == "1"


def kernel(x, w_mat):
    m_per, k = x.shape
    n = w_mat.shape[1]
    n_per = n // N_DEV
    m_glob = N_DEV * m_per
    mx = m_per // X_CHUNKS

    def body(x_hbm, w_hbm, out_hbm,
             xf_buf, x_bf, wf_buf, w_bf, send_buf, recv_buf, o_stage,
             x_sems, w_sems, send_sems, recv_sems, o_sems):
        my = lax.axis_index("i")
        sign = 1 if _UNIFORM_ORDER else 1 - 2 * (my % 2)
        dsts = [(my + (s + 1) * sign) % N_DEV for s in range(N_DEV)]
        if _UNIFORM_ORDER:
            srcs = [(my - s - 1) % N_DEV for s in range(N_DEV - 1)]
        else:
            srcs = [
                jnp.where(
                    ((my - s - 1) % N_DEV) % 2 == 0,
                    (my - s - 1) % N_DEV,
                    (my + s + 1) % N_DEV,
                )
                for s in range(N_DEV - 1)
            ]

        def out_tile_dma(slot, row):
            return pltpu.make_async_copy(
                o_stage.at[slot],
                out_hbm.at[pl.ds(row * m_per, m_per), :],
                o_sems.at[slot],
            )

        x_dmas = [
            pltpu.make_async_copy(
                x_hbm.at[pl.ds(ci * mx, mx), :], xf_buf.at[ci % 2],
                x_sems.at[ci % 2],
            )
            for ci in range(X_CHUNKS)
        ]
        w_dmas = [
            pltpu.make_async_copy(
                w_hbm.at[:, pl.ds(dsts[s] * n_per, n_per)], wf_buf.at[s % 2],
                w_sems.at[s % 2],
            )
            for s in range(N_DEV)
        ]
        if _DO_COMPUTE:
            x_dmas[0].start()
            w_dmas[0].start()
            x_dmas[1].start()
            w_dmas[1].start()

        if _DO_COMM:
            barrier_sem = pltpu.get_barrier_semaphore()
            for d in range(1, N_DEV):
                pl.semaphore_signal(
                    barrier_sem,
                    inc=1,
                    device_id=((my + d) % N_DEV,),
                    device_id_type=pl.DeviceIdType.MESH,
                )
            pl.semaphore_wait(barrier_sem, N_DEV - 1)

        for s in range(N_DEV):
            if _DO_COMPUTE:
                w_dmas[s].wait()
                w_bf[s % 2] = wf_buf[s % 2].astype(jnp.bfloat16)
                if s + 2 < N_DEV:
                    w_dmas[s + 2].start()

            for r in range(X_CHUNKS):
                if _DO_COMPUTE:
                    if s == 0:
                        x_dmas[r].wait()
                        x_bf[pl.ds(r * mx, mx), :] = (
                            xf_buf[r % 2].astype(jnp.bfloat16))
                        if r + 2 < X_CHUNKS:
                            x_dmas[r + 2].start()

                    y_sr = jnp.dot(
                        x_bf[pl.ds(r * mx, mx), :], w_bf[s % 2],
                        preferred_element_type=jnp.float32,
                    )
                    if s == N_DEV - 1:
                        o_stage[N_DEV - 1, pl.ds(r * mx, mx), :] = y_sr
                    else:
                        send_buf[s, pl.ds(r * mx, mx), :] = (
                            y_sr.astype(jnp.bfloat16))

                if _DO_COMM and s in _COMM_STEPS:
                    rdma = pltpu.make_async_remote_copy(
                        src_ref=send_buf.at[s, pl.ds(r * mx, mx), :],
                        dst_ref=recv_buf.at[s, pl.ds(r * mx, mx), :],
                        send_sem=send_sems.at[X_CHUNKS * s + r],
                        recv_sem=recv_sems.at[X_CHUNKS * s + r],
                        device_id=(dsts[s],),
                        device_id_type=pl.DeviceIdType.MESH,
                    )
                    rdma.start()

        if _DO_COMPUTE:
            out_tile_dma(N_DEV - 1, my).start()

        if _DO_COMM:
            for s in _COMM_STEPS:
                for r in range(X_CHUNKS):
                    recv = pltpu.make_async_remote_copy(
                        src_ref=send_buf.at[s, pl.ds(r * mx, mx), :],
                        dst_ref=recv_buf.at[s, pl.ds(r * mx, mx), :],
                        send_sem=send_sems.at[X_CHUNKS * s + r],
                        recv_sem=recv_sems.at[X_CHUNKS * s + r],
                        device_id=(dsts[s],),
                        device_id_type=pl.DeviceIdType.MESH,
                    )
                    recv.wait_recv()
                    o_stage[s, pl.ds(r * mx, mx), :] = (
                        recv_buf[s, pl.ds(r * mx, mx), :].astype(jnp.float32))
                out_tile_dma(s, srcs[s]).start()

            for s in _COMM_STEPS:
                for r in range(X_CHUNKS):
                    send = pltpu.make_async_remote_copy(
                        src_ref=send_buf.at[s, pl.ds(r * mx, mx), :],
                        dst_ref=recv_buf.at[s, pl.ds(r * mx, mx), :],
                        send_sem=send_sems.at[X_CHUNKS * s + r],
                        recv_sem=recv_sems.at[X_CHUNKS * s + r],
                        device_id=(dsts[s],),
                        device_id_type=pl.DeviceIdType.MESH,
                    )
                    send.wait_send()

        if _DO_COMM:
            for s in _COMM_STEPS:
                out_tile_dma(s, srcs[s]).wait()
        if _DO_COMPUTE:
            out_tile_dma(N_DEV - 1, my).wait()

    return pl.pallas_call(
        body,
        out_shape=jax.ShapeDtypeStruct((m_glob, n_per), jnp.float32),
        in_specs=[
            pl.BlockSpec(memory_space=pltpu.MemorySpace.HBM),
            pl.BlockSpec(memory_space=pltpu.MemorySpace.HBM),
        ],
        out_specs=pl.BlockSpec(memory_space=pltpu.MemorySpace.HBM),
        scratch_shapes=[
            pltpu.VMEM((2, mx, k), jnp.float32),
            pltpu.VMEM((m_per, k), jnp.bfloat16),
            pltpu.VMEM((2, k, n_per), jnp.float32),
            pltpu.VMEM((2, k, n_per), jnp.bfloat16),
            pltpu.VMEM((N_DEV - 1, m_per, n_per), jnp.bfloat16),
            pltpu.VMEM((N_DEV - 1, m_per, n_per), jnp.bfloat16),
            pltpu.VMEM((N_DEV, m_per, n_per), jnp.float32),
            pltpu.SemaphoreType.DMA((2,)),
            pltpu.SemaphoreType.DMA((2,)),
            pltpu.SemaphoreType.DMA(((N_DEV - 1) * X_CHUNKS,)),
            pltpu.SemaphoreType.DMA(((N_DEV - 1) * X_CHUNKS,)),
            pltpu.SemaphoreType.DMA((N_DEV,)),
        ],
        compiler_params=pltpu.CompilerParams(
            collective_id=0 if _DO_COMM else None,
            vmem_limit_bytes=100 * 1024 * 1024,
        ),
    )(x, w_mat)


# device time: 52202 ns/iter; 1.4905x vs baseline; 1.0019x over previous
import jax
import jax.numpy as jnp
from jax import lax
from jax.experimental import pallas as pl
from jax.experimental.pallas import tpu as pltpu

N_DEV = 4
X_CHUNKS = 4
N_STEPS = N_DEV - 1


def kernel(x, w_mat):
    m_per, k = x.shape
    n = w_mat.shape[1]
    n_per = n // N_DEV
    m_glob = N_DEV * m_per
    mx = m_per // X_CHUNKS

    def body(x_hbm, w_hbm, out_hbm,
             xf_buf, x_bf, wf_buf, w_bf, send_buf, recv_buf, o_stage,
             x_sems, w_sems, send_sems, recv_sems, o_sems):
        my = lax.axis_index("i")
        sign = 1 - 2 * (my % 2)
        dsts = [(my + (s + 1) * sign) % N_DEV for s in range(N_DEV)]
        srcs = [
            jnp.where(
                ((my - s - 1) % N_DEV) % 2 == 0,
                (my - s - 1) % N_DEV,
                (my + s + 1) % N_DEV,
            )
            for s in range(N_STEPS)
        ]

        def out_tile_dma(slot, row):
            return pltpu.make_async_copy(
                o_stage.at[slot],
                out_hbm.at[pl.ds(row * m_per, m_per), :],
                o_sems.at[slot],
            )

        def comm_rdma(s, r):
            return pltpu.make_async_remote_copy(
                src_ref=send_buf.at[s, pl.ds(r * mx, mx), :],
                dst_ref=recv_buf.at[s, pl.ds(r * mx, mx), :],
                send_sem=send_sems.at[X_CHUNKS * s + r],
                recv_sem=recv_sems.at[X_CHUNKS * s + r],
                device_id=(dsts[s],),
                device_id_type=pl.DeviceIdType.MESH,
            )

        x_dmas = [
            pltpu.make_async_copy(
                x_hbm.at[pl.ds(ci * mx, mx), :], xf_buf.at[ci % 2],
                x_sems.at[ci % 2],
            )
            for ci in range(X_CHUNKS)
        ]
        w_dmas = [
            pltpu.make_async_copy(
                w_hbm.at[:, pl.ds(dsts[s] * n_per, n_per)], wf_buf.at[s % 2],
                w_sems.at[s % 2],
            )
            for s in range(N_DEV)
        ]
        x_dmas[0].start()
        w_dmas[0].start()
        x_dmas[1].start()
        w_dmas[1].start()

        barrier_sem = pltpu.get_barrier_semaphore()
        for d in range(1, N_DEV):
            pl.semaphore_signal(
                barrier_sem,
                inc=1,
                device_id=((my + d) % N_DEV,),
                device_id_type=pl.DeviceIdType.MESH,
            )
        pl.semaphore_wait(barrier_sem, N_DEV - 1)

        for s in range(N_DEV):
            w_dmas[s].wait()
            w_bf[s % 2] = wf_buf[s % 2].astype(jnp.bfloat16)
            if s + 2 < N_DEV:
                w_dmas[s + 2].start()

            for r in range(X_CHUNKS):
                if s == 0:
                    x_dmas[r].wait()
                    x_bf[pl.ds(r * mx, mx), :] = (
                        xf_buf[r % 2].astype(jnp.bfloat16))
                    if r + 2 < X_CHUNKS:
                        x_dmas[r + 2].start()

                y_sr = jnp.dot(
                    x_bf[pl.ds(r * mx, mx), :], w_bf[s % 2],
                    preferred_element_type=jnp.float32,
                )
                if s == N_DEV - 1:
                    o_stage[N_DEV - 1, pl.ds(r * mx, mx), :] = y_sr
                else:
                    send_buf[s, pl.ds(r * mx, mx), :] = (
                        y_sr.astype(jnp.bfloat16))
                    comm_rdma(s, r).start()

        out_tile_dma(N_DEV - 1, my).start()

        for s in range(N_STEPS):
            for r in range(X_CHUNKS):
                comm_rdma(s, r).wait_recv()
                o_stage[s, pl.ds(r * mx, mx), :] = (
                    recv_buf[s, pl.ds(r * mx, mx), :].astype(jnp.float32))
            out_tile_dma(s, srcs[s]).start()

        for s in range(N_STEPS):
            for r in range(X_CHUNKS):
                comm_rdma(s, r).wait_send()

        for s in range(N_STEPS):
            out_tile_dma(s, srcs[s]).wait()
        out_tile_dma(N_DEV - 1, my).wait()

    return pl.pallas_call(
        body,
        out_shape=jax.ShapeDtypeStruct((m_glob, n_per), jnp.float32),
        in_specs=[
            pl.BlockSpec(memory_space=pltpu.MemorySpace.HBM),
            pl.BlockSpec(memory_space=pltpu.MemorySpace.HBM),
        ],
        out_specs=pl.BlockSpec(memory_space=pltpu.MemorySpace.HBM),
        scratch_shapes=[
            pltpu.VMEM((2, mx, k), jnp.float32),
            pltpu.VMEM((m_per, k), jnp.bfloat16),
            pltpu.VMEM((2, k, n_per), jnp.float32),
            pltpu.VMEM((2, k, n_per), jnp.bfloat16),
            pltpu.VMEM((N_STEPS, m_per, n_per), jnp.bfloat16),
            pltpu.VMEM((N_STEPS, m_per, n_per), jnp.bfloat16),
            pltpu.VMEM((N_DEV, m_per, n_per), jnp.float32),
            pltpu.SemaphoreType.DMA((2,)),
            pltpu.SemaphoreType.DMA((2,)),
            pltpu.SemaphoreType.DMA((N_STEPS * X_CHUNKS,)),
            pltpu.SemaphoreType.DMA((N_STEPS * X_CHUNKS,)),
            pltpu.SemaphoreType.DMA((N_DEV,)),
        ],
        compiler_params=pltpu.CompilerParams(
            collective_id=0,
            vmem_limit_bytes=100 * 1024 * 1024,
        ),
    )(x, w_mat)
